# baseline (speedup 1.0000x reference)
import functools

import jax
import jax.numpy as jnp
from jax import lax
from jax.experimental import pallas as pl
from jax.experimental.pallas import tpu as pltpu
from jax.experimental.pallas import tpu_sc as plsc

B = 4096
F = 26
V = 100000
D = 6
N = B * F
FV = F * V
CH = 128
NW = 32
NCH = N // (NW * CH)
NPW = NCH * CH
TEMP = 0.5
BN_EPS = 1e-3
TARGET_VEC_SIZES = (1, 2, 4, 6)


def _sc_gather(idx6, emb_1d, bias_1d):
    mesh = plsc.VectorSubcoreMesh(core_axis_name="c", subcore_axis_name="s")

    @functools.partial(
        pl.kernel,
        out_type=[
            jax.ShapeDtypeStruct((D, N), jnp.float32),
            jax.ShapeDtypeStruct((N,), jnp.float32),
        ],
        mesh=mesh,
        scratch_types=[
            pltpu.VMEM((D, NPW), jnp.int32),
            pltpu.VMEM((D, NPW), jnp.float32),
            pltpu.VMEM((NPW,), jnp.float32),
            pltpu.SemaphoreType.DMA,
            pltpu.SemaphoreType.DMA,
        ],
        compiler_params=pltpu.CompilerParams(use_tc_tiling_on_sc=False),
    )
    def k(idx_hbm, emb_hbm, bias_hbm, out_emb, out_bias,
          idx_v, rows_v, brows_v, sem_e, sem_b):
        w = lax.axis_index("s") * 2 + lax.axis_index("c")
        base = w * NPW
        for d in range(D):
            pltpu.sync_copy(idx_hbm.at[d, pl.ds(base, NPW)], idx_v.at[d])
        cps = []
        for j in range(NCH):
            sl = pl.ds(j * CH, CH)
            for d in range(D):
                cps.append(pltpu.async_copy(
                    emb_hbm.at[idx_v.at[d, sl]], rows_v.at[d, sl], sem_e))
            cps.append(pltpu.async_copy(
                bias_hbm.at[idx_v.at[0, sl]], brows_v.at[sl], sem_b))
        for cp in cps:
            cp.wait()
        for d in range(D):
            pltpu.sync_copy(rows_v.at[d], out_emb.at[d, pl.ds(base, NPW)])
        pltpu.sync_copy(brows_v, out_bias.at[pl.ds(base, NPW)])

    return k(idx6, emb_1d, bias_1d)


def _choice_row(p):
    ki = lax.broadcasted_iota(jnp.int32, (len(TARGET_VEC_SIZES), F * D), 0)
    jd = lax.broadcasted_iota(jnp.int32, (len(TARGET_VEC_SIZES), F * D), 1) % D
    sel = jnp.where(jd < 2, jd, jnp.where(jd < 4, 2, 3))
    K = (sel == ki).astype(jnp.float32)
    S = jnp.dot(p, K, preferred_element_type=jnp.float32)
    fi = lax.broadcasted_iota(jnp.int32, (F, F * D), 0)
    jf = lax.broadcasted_iota(jnp.int32, (F, F * D), 1) // D
    E = (fi == jf).astype(jnp.float32)
    return jnp.sum(S * E, axis=0, keepdims=True)


def _tc_body(x_ref, bv_ref, nas_ref, W1_ref, b1_ref, W2_ref, b2_ref,
             W3_ref, b3_ref, o_ref):
    x = x_ref[...]
    mean = jnp.mean(x, axis=0, keepdims=True)
    var = jnp.mean(x * x, axis=0, keepdims=True) - mean * mean
    inv = lax.rsqrt(var + BN_EPS)
    logits = nas_ref[...] * (1.0 / TEMP)
    m = jnp.max(logits, axis=1, keepdims=True)
    e = jnp.exp(logits - m)
    p = e / jnp.sum(e, axis=1, keepdims=True)
    c = _choice_row(p)
    xs = (x - mean) * (c * inv)
    h = jnp.dot(xs, W1_ref[...], preferred_element_type=jnp.float32)
    h = jnp.maximum(h + b1_ref[...], 0.0)
    h = jnp.dot(h, W2_ref[...], preferred_element_type=jnp.float32)
    h = jnp.maximum(h + b2_ref[...], 0.0)
    o = jnp.dot(h, W3_ref[...], preferred_element_type=jnp.float32)
    o = o + b3_ref[...]
    bsum = jnp.sum(bv_ref[...], axis=1, keepdims=True)
    o_ref[...] = jax.nn.sigmoid(o + bsum)


def _tc_mlp(x, bv, nas, W1, b1, W2, b2, W3, b3):
    return pl.pallas_call(
        _tc_body,
        out_shape=jax.ShapeDtypeStruct((B, 1), jnp.float32),
    )(x, bv, nas, W1, b1, W2, b2, W3, b3)


def kernel(inputs, emb_table, bias_table, nas_logits, W1, b1, W2, b2, W3, b3):
    inputs = inputs.astype(jnp.int32)
    offs = (jnp.arange(F, dtype=jnp.int32) * V)[None, :]
    base_idx = (inputs + offs).reshape(1, N)
    doffs = (jnp.arange(D, dtype=jnp.int32) * FV)[:, None]
    idx6 = base_idx + doffs
    emb_1d = emb_table.transpose(2, 0, 1).reshape(FV * D)
    bias_1d = bias_table.reshape(FV)
    rows6, brows = _sc_gather(idx6, emb_1d, bias_1d)
    x = rows6.reshape(D, B, F).transpose(1, 2, 0).reshape(B, F * D)
    bv = brows.reshape(B, F)
    out = _tc_mlp(x, bv, nas_logits, W1, b1.reshape(1, -1), W2,
                  b2.reshape(1, -1), W3, b3.reshape(1, -1))
    return out.reshape(B)
